# Initial kernel scaffold; baseline (speedup 1.0000x reference)
import functools

import jax
import jax.numpy as jnp
from jax import lax
from jax.experimental import pallas as pl
from jax.experimental.pallas import tpu as pltpu
from jax.experimental.pallas import tpu_sc as plsc

NC = 2
NS = 16
LANES = 16
NW = NC * NS
CH = 128
RB = 1024
ZR = 160


def _sc_mesh():
    return plsc.VectorSubcoreMesh(core_axis_name="c", subcore_axis_name="s")


@functools.lru_cache(maxsize=None)
def _hist_kernel(n_pad, n_chunks):
    @functools.partial(
        pl.kernel,
        out_type=jax.ShapeDtypeStruct((NW, n_pad), jnp.float32),
        mesh=_sc_mesh(),
        scratch_types=[
            pltpu.VMEM((n_chunks, CH), jnp.int32),
            pltpu.VMEM((n_pad,), jnp.float32),
        ],
    )
    def hist(dsts_hbm, zeros1_hbm, out_hbm, idx_v, hist_v):
        c = lax.axis_index("c")
        s = lax.axis_index("s")
        w = c * NS + s
        pltpu.sync_copy(dsts_hbm.at[w], idx_v)
        pltpu.sync_copy(zeros1_hbm, hist_v)
        ones = jnp.ones((LANES,), jnp.float32)

        def body(g, carry):
            for j in range(CH // LANES):
                idx = idx_v[g, pl.ds(j * LANES, LANES)]
                plsc.addupdate_scatter(hist_v, [idx], ones)
            return carry

        lax.fori_loop(0, n_chunks, body, 0)
        pltpu.sync_copy(hist_v, out_hbm.at[w])

    return hist


@functools.lru_cache(maxsize=None)
def _prop_kernel(n_pad, n_chunks):
    rows_per_tile = n_pad // NS

    @functools.partial(
        pl.kernel,
        out_type=jax.ShapeDtypeStruct((NC, n_pad, 128), jnp.float32),
        mesh=_sc_mesh(),
        scratch_types=[
            pltpu.VMEM((n_chunks, CH), jnp.int32),
            pltpu.VMEM((n_chunks, CH), jnp.int32),
            pltpu.VMEM((CH, 128), jnp.float32),
            pltpu.VMEM((CH, 128), jnp.float32),
            pltpu.VMEM((ZR, 128), jnp.float32),
            pltpu.VMEM_SHARED((n_pad, 128), jnp.float32),
            pltpu.SemaphoreType.DMA,
            pltpu.SemaphoreType.DMA,
        ],
    )
    def prop(h_hbm, srcs_hbm, dsts_hbm, zrows_hbm, out_hbm,
             src_v, dst_v, buf0, buf1, zrow_v, acc_sh, sem0, sem1):
        c = lax.axis_index("c")
        s = lax.axis_index("s")
        w = c * NS + s
        pltpu.sync_copy(srcs_hbm.at[w], src_v)
        pltpu.sync_copy(dsts_hbm.at[w], dst_v)
        pltpu.sync_copy(zrows_hbm, zrow_v)
        base = s * rows_per_tile
        for k in range(rows_per_tile // ZR):
            pltpu.sync_copy(zrow_v, acc_sh.at[pl.ds(base + k * ZR, ZR)])
        plsc.subcore_barrier()

        pltpu.make_async_copy(h_hbm.at[src_v.at[0]], buf0, sem0).start()
        pltpu.make_async_copy(h_hbm.at[src_v.at[1]], buf1, sem1).start()

        def body(i, carry):
            g = 2 * i
            pltpu.make_async_copy(h_hbm.at[src_v.at[g]], buf0, sem0).wait()
            pltpu.sync_copy(buf0, acc_sh.at[dst_v.at[g]], add=True)
            pltpu.make_async_copy(h_hbm.at[src_v.at[g + 2]], buf0, sem0).start()
            pltpu.make_async_copy(h_hbm.at[src_v.at[g + 1]], buf1, sem1).wait()
            pltpu.sync_copy(buf1, acc_sh.at[dst_v.at[g + 1]], add=True)
            pltpu.make_async_copy(h_hbm.at[src_v.at[g + 3]], buf1, sem1).start()
            return carry

        lax.fori_loop(0, n_chunks // 2 - 1, body, 0)
        g_last = n_chunks - 2
        pltpu.make_async_copy(h_hbm.at[src_v.at[g_last]], buf0, sem0).wait()
        pltpu.sync_copy(buf0, acc_sh.at[dst_v.at[g_last]], add=True)
        pltpu.make_async_copy(h_hbm.at[src_v.at[g_last + 1]], buf1, sem1).wait()
        pltpu.sync_copy(buf1, acc_sh.at[dst_v.at[g_last + 1]], add=True)
        plsc.subcore_barrier()

        for k in range(rows_per_tile // CH):
            r = base + k * CH
            pltpu.sync_copy(acc_sh.at[pl.ds(r, CH)], buf0)
            pltpu.sync_copy(buf0, out_hbm.at[c, pl.ds(r, CH)])

    return prop


def _dinv(hist):
    nw, n_pad = hist.shape

    def body(hist_ref, out_ref):
        deg = 1.0 + jnp.sum(hist_ref[...], axis=0)
        out_ref[...] = lax.rsqrt(deg)

    return pl.pallas_call(
        body,
        grid=(n_pad // RB,),
        in_specs=[pl.BlockSpec((nw, RB), lambda i: (0, i))],
        out_specs=pl.BlockSpec((RB,), lambda i: (i,)),
        out_shape=jax.ShapeDtypeStruct((n_pad,), jnp.float32),
    )(hist)


def _mm_scale(x, W, dinv2):
    n_pad, d_in = x.shape
    d_out = W.shape[1]

    def body(x_ref, w_ref, dv_ref, out_ref):
        h = jnp.dot(x_ref[...], w_ref[...], preferred_element_type=jnp.float32)
        out_ref[...] = h * dv_ref[...]

    return pl.pallas_call(
        body,
        grid=(n_pad // RB,),
        in_specs=[
            pl.BlockSpec((RB, d_in), lambda i: (i, 0)),
            pl.BlockSpec((d_in, d_out), lambda i: (0, 0)),
            pl.BlockSpec((RB, 1), lambda i: (i, 0)),
        ],
        out_specs=pl.BlockSpec((RB, d_out), lambda i: (i, 0)),
        out_shape=jax.ShapeDtypeStruct((n_pad, d_out), jnp.float32),
    )(x, W, dinv2)


def _mid_layer(p0, p1, hh, dinv2, b, W):
    n_pad, d = hh.shape
    d_out = W.shape[1]

    def body(p0_ref, p1_ref, hh_ref, dv_ref, b_ref, w_ref, out_ref):
        z = dv_ref[...] * (p0_ref[...] + p1_ref[...] + hh_ref[...]) + b_ref[...]
        h1 = jnp.maximum(z, 0.0)
        h = jnp.dot(h1, w_ref[...], preferred_element_type=jnp.float32)
        out_ref[...] = h * dv_ref[...]

    return pl.pallas_call(
        body,
        grid=(n_pad // RB,),
        in_specs=[
            pl.BlockSpec((RB, d), lambda i: (i, 0)),
            pl.BlockSpec((RB, d), lambda i: (i, 0)),
            pl.BlockSpec((RB, d), lambda i: (i, 0)),
            pl.BlockSpec((RB, 1), lambda i: (i, 0)),
            pl.BlockSpec((d,), lambda i: (0,)),
            pl.BlockSpec((d, d_out), lambda i: (0, 0)),
        ],
        out_specs=pl.BlockSpec((RB, d_out), lambda i: (i, 0)),
        out_shape=jax.ShapeDtypeStruct((n_pad, d_out), jnp.float32),
    )(p0, p1, hh, dinv2, b, W)


def _final_layer(q0, q1, hh, dinv2, b):
    n_pad, d = hh.shape

    def body(q0_ref, q1_ref, hh_ref, dv_ref, b_ref, out_ref):
        z = dv_ref[...] * (q0_ref[...] + q1_ref[...] + hh_ref[...]) + b_ref[...]
        m = jnp.max(z, axis=1, keepdims=True)
        zz = z - m
        out_ref[...] = zz - jnp.log(jnp.sum(jnp.exp(zz), axis=1, keepdims=True))

    return pl.pallas_call(
        body,
        grid=(n_pad // RB,),
        in_specs=[
            pl.BlockSpec((RB, d), lambda i: (i, 0)),
            pl.BlockSpec((RB, d), lambda i: (i, 0)),
            pl.BlockSpec((RB, d), lambda i: (i, 0)),
            pl.BlockSpec((RB, 1), lambda i: (i, 0)),
            pl.BlockSpec((d,), lambda i: (0,)),
        ],
        out_specs=pl.BlockSpec((RB, d), lambda i: (i, 0)),
        out_shape=jax.ShapeDtypeStruct((n_pad, d), jnp.float32),
    )(q0, q1, hh, dinv2, b)


def kernel(x, edge_index, W1, b1, W2, b2):
    N, d_in = x.shape
    E = edge_index.shape[1]
    n_pad = (N // RB + 1) * RB
    n_chunks = -(-E // (NW * CH))
    if n_chunks % 2:
        n_chunks += 1
    e_pad = NW * CH * n_chunks

    src = edge_index[0].astype(jnp.int32)
    dst = edge_index[1].astype(jnp.int32)
    pad = e_pad - E
    srcs = jnp.concatenate([src, jnp.zeros((pad,), jnp.int32)])
    dsts = jnp.concatenate([dst, jnp.full((pad,), N, jnp.int32)])
    srcs = srcs.reshape(NW, n_chunks, CH)
    dsts = dsts.reshape(NW, n_chunks, CH)
    x_pad = jnp.pad(x, ((0, n_pad - N), (0, 0)))
    zrows = jnp.zeros((ZR, 128), jnp.float32)
    zeros1 = jnp.zeros((n_pad,), jnp.float32)

    hist = _hist_kernel(n_pad, n_chunks)(dsts, zeros1)
    dinv2 = _dinv(hist)[:, None]

    prop = _prop_kernel(n_pad, n_chunks)
    h1h = _mm_scale(x_pad, W1, dinv2)
    P = prop(h1h, srcs, dsts, zrows)
    h2h = _mid_layer(P[0], P[1], h1h, dinv2, b1, W2)
    Q = prop(h2h, srcs, dsts, zrows)
    out = _final_layer(Q[0], Q[1], h2h, dinv2, b2)
    return out[:N]


# speedup vs baseline: 8.1706x; 8.1706x over previous
import functools

import jax
import jax.numpy as jnp
from jax import lax
from jax.experimental import pallas as pl
from jax.experimental.pallas import tpu as pltpu
from jax.experimental.pallas import tpu_sc as plsc

NC = 2
NS = 16
LANES = 16
NW = NC * NS
CH = 128
RB = 1024


def _sc_mesh():
    return plsc.VectorSubcoreMesh(core_axis_name="c", subcore_axis_name="s")


@functools.lru_cache(maxsize=None)
def _hist_kernel(n_pad, n_chunks):
    @functools.partial(
        pl.kernel,
        out_type=jax.ShapeDtypeStruct((NW, n_pad), jnp.float32),
        mesh=_sc_mesh(),
        scratch_types=[
            pltpu.VMEM((n_chunks, CH), jnp.int32),
            pltpu.VMEM((n_pad,), jnp.float32),
        ],
        compiler_params=pltpu.CompilerParams(needs_layout_passes=False),
    )
    def hist(dsts_hbm, zeros1_hbm, out_hbm, idx_v, hist_v):
        c = lax.axis_index("c")
        s = lax.axis_index("s")
        w = c * NS + s
        pltpu.sync_copy(dsts_hbm.at[w], idx_v)
        pltpu.sync_copy(zeros1_hbm, hist_v)
        ones = jnp.ones((LANES,), jnp.float32)

        def body(g, carry):
            for j in range(CH // LANES):
                idx = idx_v[g, pl.ds(j * LANES, LANES)]
                plsc.addupdate_scatter(hist_v, [idx], ones)
            return carry

        lax.fori_loop(0, n_chunks, body, 0)
        pltpu.sync_copy(hist_v, out_hbm.at[w])

    return hist


@functools.lru_cache(maxsize=None)
def _prop_kernel(n_pad, n_chunks):
    rows_per_tile = n_pad // NS

    @functools.partial(
        pl.kernel,
        out_type=jax.ShapeDtypeStruct((NC, n_pad, 128), jnp.float32),
        mesh=_sc_mesh(),
        scratch_types=[
            pltpu.VMEM((CH,), jnp.int32),
            pltpu.VMEM((CH,), jnp.int32),
            pltpu.VMEM((CH,), jnp.int32),
            pltpu.VMEM((CH,), jnp.int32),
            pltpu.VMEM((CH, 128), jnp.float32),
            pltpu.VMEM((CH, 128), jnp.float32),
            pltpu.VMEM_SHARED((n_pad, 128), jnp.float32),
            pltpu.SemaphoreType.DMA,
            pltpu.SemaphoreType.DMA,
            pltpu.SemaphoreType.DMA,
            pltpu.SemaphoreType.DMA,
        ],
        compiler_params=pltpu.CompilerParams(needs_layout_passes=False),
    )
    def prop(h_hbm, srcs_hbm, dsts_hbm, zrows_hbm, out_hbm,
             srcb0, srcb1, dstb0, dstb1, buf0, buf1, acc_sh,
             semi0, semi1, semr0, semr1):
        c = lax.axis_index("c")
        s = lax.axis_index("s")
        w = c * NS + s
        srcb = (srcb0, srcb1)
        dstb = (dstb0, dstb1)
        bufs = (buf0, buf1)
        semi = (semi0, semi1)
        semr = (semr0, semr1)

        def start_idx(g, p):
            pltpu.make_async_copy(srcs_hbm.at[w, g], srcb[p], semi[p]).start()
            pltpu.make_async_copy(dsts_hbm.at[w, g], dstb[p], semi[p]).start()

        def wait_idx(g, p):
            pltpu.make_async_copy(srcs_hbm.at[w, g], srcb[p], semi[p]).wait()
            pltpu.make_async_copy(dsts_hbm.at[w, g], dstb[p], semi[p]).wait()

        pltpu.sync_copy(zrows_hbm, buf0)
        base = s * rows_per_tile
        for k in range(rows_per_tile // CH):
            pltpu.sync_copy(buf0, acc_sh.at[pl.ds(base + k * CH, CH)])
        plsc.subcore_barrier()

        start_idx(0, 0)
        start_idx(1, 1)
        wait_idx(0, 0)
        pltpu.make_async_copy(h_hbm.at[srcb[0]], bufs[0], semr[0]).start()

        def step(g, p):
            pp = 1 - p
            pltpu.make_async_copy(h_hbm.at[srcb[p]], bufs[p], semr[p]).wait()
            pltpu.sync_copy(bufs[p], acc_sh.at[dstb[p]], add=True)

            @pl.when(g + 2 < n_chunks)
            def _():
                start_idx(g + 2, p)

            wait_idx(g + 1, pp)
            pltpu.make_async_copy(h_hbm.at[srcb[pp]], bufs[pp], semr[pp]).start()

        def body(i, carry):
            g = 2 * i
            step(g, 0)
            step(g + 1, 1)
            return carry

        lax.fori_loop(0, n_chunks // 2 - 1, body, 0)
        pltpu.make_async_copy(h_hbm.at[srcb[0]], bufs[0], semr[0]).wait()
        pltpu.sync_copy(bufs[0], acc_sh.at[dstb[0]], add=True)
        wait_idx(n_chunks - 1, 1)
        pltpu.make_async_copy(h_hbm.at[srcb[1]], bufs[1], semr[1]).start()
        pltpu.make_async_copy(h_hbm.at[srcb[1]], bufs[1], semr[1]).wait()
        pltpu.sync_copy(bufs[1], acc_sh.at[dstb[1]], add=True)
        plsc.subcore_barrier()

        for k in range(rows_per_tile // CH):
            r = base + k * CH
            pltpu.sync_copy(acc_sh.at[pl.ds(r, CH)], buf0)
            pltpu.sync_copy(buf0, out_hbm.at[c, pl.ds(r, CH)])

    return prop


def _dinv(hist):
    nw, n_pad = hist.shape

    def body(hist_ref, out_ref):
        deg = 1.0 + jnp.sum(hist_ref[...], axis=0)
        out_ref[...] = lax.rsqrt(deg)

    return pl.pallas_call(
        body,
        grid=(n_pad // RB,),
        in_specs=[pl.BlockSpec((nw, RB), lambda i: (0, i))],
        out_specs=pl.BlockSpec((RB,), lambda i: (i,)),
        out_shape=jax.ShapeDtypeStruct((n_pad,), jnp.float32),
    )(hist)


def _mm_scale(x, W, dinv2):
    n_pad, d_in = x.shape
    d_out = W.shape[1]

    def body(x_ref, w_ref, dv_ref, out_ref):
        h = jnp.dot(x_ref[...], w_ref[...], preferred_element_type=jnp.float32)
        out_ref[...] = h * dv_ref[...]

    return pl.pallas_call(
        body,
        grid=(n_pad // RB,),
        in_specs=[
            pl.BlockSpec((RB, d_in), lambda i: (i, 0)),
            pl.BlockSpec((d_in, d_out), lambda i: (0, 0)),
            pl.BlockSpec((RB, 1), lambda i: (i, 0)),
        ],
        out_specs=pl.BlockSpec((RB, d_out), lambda i: (i, 0)),
        out_shape=jax.ShapeDtypeStruct((n_pad, d_out), jnp.float32),
    )(x, W, dinv2)


def _mid_layer(p0, p1, hh, dinv2, b, W):
    n_pad, d = hh.shape
    d_out = W.shape[1]

    def body(p0_ref, p1_ref, hh_ref, dv_ref, b_ref, w_ref, out_ref):
        z = dv_ref[...] * (p0_ref[...] + p1_ref[...] + hh_ref[...]) + b_ref[...]
        h1 = jnp.maximum(z, 0.0)
        h = jnp.dot(h1, w_ref[...], preferred_element_type=jnp.float32)
        out_ref[...] = h * dv_ref[...]

    return pl.pallas_call(
        body,
        grid=(n_pad // RB,),
        in_specs=[
            pl.BlockSpec((RB, d), lambda i: (i, 0)),
            pl.BlockSpec((RB, d), lambda i: (i, 0)),
            pl.BlockSpec((RB, d), lambda i: (i, 0)),
            pl.BlockSpec((RB, 1), lambda i: (i, 0)),
            pl.BlockSpec((d,), lambda i: (0,)),
            pl.BlockSpec((d, d_out), lambda i: (0, 0)),
        ],
        out_specs=pl.BlockSpec((RB, d_out), lambda i: (i, 0)),
        out_shape=jax.ShapeDtypeStruct((n_pad, d_out), jnp.float32),
    )(p0, p1, hh, dinv2, b, W)


def _final_layer(q0, q1, hh, dinv2, b):
    n_pad, d = hh.shape

    def body(q0_ref, q1_ref, hh_ref, dv_ref, b_ref, out_ref):
        z = dv_ref[...] * (q0_ref[...] + q1_ref[...] + hh_ref[...]) + b_ref[...]
        m = jnp.max(z, axis=1, keepdims=True)
        zz = z - m
        out_ref[...] = zz - jnp.log(jnp.sum(jnp.exp(zz), axis=1, keepdims=True))

    return pl.pallas_call(
        body,
        grid=(n_pad // RB,),
        in_specs=[
            pl.BlockSpec((RB, d), lambda i: (i, 0)),
            pl.BlockSpec((RB, d), lambda i: (i, 0)),
            pl.BlockSpec((RB, d), lambda i: (i, 0)),
            pl.BlockSpec((RB, 1), lambda i: (i, 0)),
            pl.BlockSpec((d,), lambda i: (0,)),
        ],
        out_specs=pl.BlockSpec((RB, d), lambda i: (i, 0)),
        out_shape=jax.ShapeDtypeStruct((n_pad, d), jnp.float32),
    )(q0, q1, hh, dinv2, b)


def kernel(x, edge_index, W1, b1, W2, b2):
    N, d_in = x.shape
    E = edge_index.shape[1]
    n_pad = (N // RB + 1) * RB
    n_chunks = -(-E // (NW * CH))
    if n_chunks % 2:
        n_chunks += 1
    e_pad = NW * CH * n_chunks

    src = edge_index[0].astype(jnp.int32)
    dst = edge_index[1].astype(jnp.int32)
    pad = e_pad - E
    srcs = jnp.concatenate([src, jnp.zeros((pad,), jnp.int32)])
    dsts = jnp.concatenate([dst, jnp.full((pad,), N, jnp.int32)])
    srcs = srcs.reshape(NW, n_chunks, CH)
    dsts = dsts.reshape(NW, n_chunks, CH)
    x_pad = jnp.pad(x, ((0, n_pad - N), (0, 0)))
    zrows = jnp.zeros((CH, 128), jnp.float32)
    zeros1 = jnp.zeros((n_pad,), jnp.float32)

    hist = _hist_kernel(n_pad, n_chunks)(dsts, zeros1)
    dinv2 = _dinv(hist)[:, None]

    prop = _prop_kernel(n_pad, n_chunks)
    h1h = _mm_scale(x_pad, W1, dinv2)
    P = prop(h1h, srcs, dsts, zrows)
    h2h = _mid_layer(P[0], P[1], h1h, dinv2, b1, W2)
    Q = prop(h2h, srcs, dsts, zrows)
    out = _final_layer(Q[0], Q[1], h2h, dinv2, b2)
    return out[:N]


# speedup vs baseline: 17.7479x; 2.1722x over previous
import functools

import jax
import jax.numpy as jnp
from jax import lax
from jax.experimental import pallas as pl
from jax.experimental.pallas import tpu as pltpu
from jax.experimental.pallas import tpu_sc as plsc

NC = 2
NS = 16
LANES = 16
NW = NC * NS
CH = 120
CHH = 128
NSL = 3
NIS = 6
RB = 1024


def _sc_mesh():
    return plsc.VectorSubcoreMesh(core_axis_name="c", subcore_axis_name="s")


@functools.lru_cache(maxsize=None)
def _hist_kernel(n_pad, n_chunks):
    @functools.partial(
        pl.kernel,
        out_type=jax.ShapeDtypeStruct((NW, n_pad), jnp.float32),
        mesh=_sc_mesh(),
        scratch_types=[
            pltpu.VMEM((n_chunks, CHH), jnp.int32),
            pltpu.VMEM((n_pad,), jnp.float32),
        ],
        compiler_params=pltpu.CompilerParams(needs_layout_passes=False),
    )
    def hist(dsts_hbm, zeros1_hbm, out_hbm, idx_v, hist_v):
        c = lax.axis_index("c")
        s = lax.axis_index("s")
        w = c * NS + s
        pltpu.sync_copy(dsts_hbm.at[w], idx_v)
        pltpu.sync_copy(zeros1_hbm, hist_v)
        ones = jnp.ones((LANES,), jnp.float32)

        def body(g, carry):
            for j in range(CHH // LANES):
                idx = idx_v[g, pl.ds(j * LANES, LANES)]
                plsc.addupdate_scatter(hist_v, [idx], ones)
            return carry

        lax.fori_loop(0, n_chunks, body, 0)
        pltpu.sync_copy(hist_v, out_hbm.at[w])

    return hist


@functools.lru_cache(maxsize=None)
def _prop_kernel(n_pad, n_chunks):
    rows_per_tile = n_pad // NS
    wb_sizes = [CH] * (rows_per_tile // CH)
    if rows_per_tile % CH:
        wb_sizes.append(rows_per_tile % CH)

    @functools.partial(
        pl.kernel,
        out_type=jax.ShapeDtypeStruct((NC, n_pad, 128), jnp.float32),
        mesh=_sc_mesh(),
        scratch_types=(
            [pltpu.VMEM((CH,), jnp.int32) for _ in range(2 * NIS)]
            + [pltpu.VMEM((CH, 128), jnp.float32) for _ in range(NSL)]
            + [pltpu.VMEM_SHARED((n_pad, 128), jnp.float32)]
            + [pltpu.SemaphoreType.DMA] * (NIS + 2 * NSL)
        ),
        compiler_params=pltpu.CompilerParams(needs_layout_passes=False),
    )
    def prop(h_hbm, srcs_hbm, dsts_hbm, zrows_hbm, out_hbm, *scr):
        srcb = scr[0:NIS]
        dstb = scr[NIS:2 * NIS]
        bufs = scr[2 * NIS:2 * NIS + NSL]
        acc_sh = scr[2 * NIS + NSL]
        semi = scr[2 * NIS + NSL + 1:2 * NIS + NSL + 1 + NIS]
        semg = scr[2 * NIS + NSL + 1 + NIS:2 * NIS + NSL + 1 + NIS + NSL]
        semw = scr[2 * NIS + NSL + 1 + NIS + NSL:]
        c = lax.axis_index("c")
        s = lax.axis_index("s")
        w = c * NS + s

        def start_idx(g, k):
            pltpu.make_async_copy(srcs_hbm.at[w, g], srcb[k], semi[k]).start()
            pltpu.make_async_copy(dsts_hbm.at[w, g], dstb[k], semi[k]).start()

        def wait_idx(g, k):
            pltpu.make_async_copy(srcs_hbm.at[w, g], srcb[k], semi[k]).wait()
            pltpu.make_async_copy(dsts_hbm.at[w, g], dstb[k], semi[k]).wait()

        def start_gather(p, k):
            pltpu.make_async_copy(h_hbm.at[srcb[k]], bufs[p], semg[p]).start()

        def wait_gather(p):
            pltpu.make_async_copy(h_hbm.at[srcb[0]], bufs[p], semg[p]).wait()

        def start_scatter(p, k):
            pltpu.async_copy(bufs[p], acc_sh.at[dstb[k]], semw[p], add=True)

        def wait_scatter(p):
            pltpu.make_async_copy(bufs[p], acc_sh.at[dstb[0]], semw[p]).wait()

        pltpu.sync_copy(zrows_hbm, bufs[0])
        base = s * rows_per_tile
        off = 0
        for sz in wb_sizes:
            pltpu.sync_copy(bufs[0].at[pl.ds(0, sz)],
                            acc_sh.at[pl.ds(base + off, sz)])
            off += sz
        plsc.subcore_barrier()

        for k in range(min(NIS, n_chunks)):
            start_idx(k, k)
        wait_idx(0, 0)
        start_gather(0, 0)
        wait_idx(1, 1)
        start_gather(1, 1)

        def body(i, carry):
            for j in range(NIS):
                g = NIS * i + j
                p = j % NSL
                q = (j + 2) % NSL
                ki = (j + 2) % NIS
                kr = (j + NIS - 1) % NIS
                wait_gather(p)
                start_scatter(p, j)

                @pl.when(g >= 1)
                def _():
                    wait_scatter(q)

                @pl.when((g >= 1) & (g + NIS - 1 < n_chunks))
                def _():
                    start_idx(g + NIS - 1, kr)

                @pl.when(g + 2 < n_chunks)
                def _():
                    wait_idx(g + 2, ki)
                    start_gather(q, ki)

            return carry

        lax.fori_loop(0, n_chunks // NIS, body, 0)
        wait_scatter((n_chunks - 1) % NSL)
        plsc.subcore_barrier()

        off = 0
        for sz in wb_sizes:
            r = base + off
            pltpu.sync_copy(acc_sh.at[pl.ds(r, sz)], bufs[0].at[pl.ds(0, sz)])
            pltpu.sync_copy(bufs[0].at[pl.ds(0, sz)],
                            out_hbm.at[c, pl.ds(r, sz)])
            off += sz

    return prop


def _dinv(hist):
    nw, n_pad = hist.shape

    def body(hist_ref, out_ref):
        deg = 1.0 + jnp.sum(hist_ref[...], axis=0)
        out_ref[...] = lax.rsqrt(deg)

    return pl.pallas_call(
        body,
        grid=(n_pad // RB,),
        in_specs=[pl.BlockSpec((nw, RB), lambda i: (0, i))],
        out_specs=pl.BlockSpec((RB,), lambda i: (i,)),
        out_shape=jax.ShapeDtypeStruct((n_pad,), jnp.float32),
    )(hist)


def _mm_scale(x, W, dinv2):
    n_pad, d_in = x.shape
    d_out = W.shape[1]

    def body(x_ref, w_ref, dv_ref, out_ref):
        h = jnp.dot(x_ref[...], w_ref[...], preferred_element_type=jnp.float32)
        out_ref[...] = h * dv_ref[...]

    return pl.pallas_call(
        body,
        grid=(n_pad // RB,),
        in_specs=[
            pl.BlockSpec((RB, d_in), lambda i: (i, 0)),
            pl.BlockSpec((d_in, d_out), lambda i: (0, 0)),
            pl.BlockSpec((RB, 1), lambda i: (i, 0)),
        ],
        out_specs=pl.BlockSpec((RB, d_out), lambda i: (i, 0)),
        out_shape=jax.ShapeDtypeStruct((n_pad, d_out), jnp.float32),
    )(x, W, dinv2)


def _mid_layer(p0, p1, hh, dinv2, b, W):
    n_pad, d = hh.shape
    d_out = W.shape[1]

    def body(p0_ref, p1_ref, hh_ref, dv_ref, b_ref, w_ref, out_ref):
        z = dv_ref[...] * (p0_ref[...] + p1_ref[...] + hh_ref[...]) + b_ref[...]
        h1 = jnp.maximum(z, 0.0)
        h = jnp.dot(h1, w_ref[...], preferred_element_type=jnp.float32)
        out_ref[...] = h * dv_ref[...]

    return pl.pallas_call(
        body,
        grid=(n_pad // RB,),
        in_specs=[
            pl.BlockSpec((RB, d), lambda i: (i, 0)),
            pl.BlockSpec((RB, d), lambda i: (i, 0)),
            pl.BlockSpec((RB, d), lambda i: (i, 0)),
            pl.BlockSpec((RB, 1), lambda i: (i, 0)),
            pl.BlockSpec((d,), lambda i: (0,)),
            pl.BlockSpec((d, d_out), lambda i: (0, 0)),
        ],
        out_specs=pl.BlockSpec((RB, d_out), lambda i: (i, 0)),
        out_shape=jax.ShapeDtypeStruct((n_pad, d_out), jnp.float32),
    )(p0, p1, hh, dinv2, b, W)


def _final_layer(q0, q1, hh, dinv2, b):
    n_pad, d = hh.shape

    def body(q0_ref, q1_ref, hh_ref, dv_ref, b_ref, out_ref):
        z = dv_ref[...] * (q0_ref[...] + q1_ref[...] + hh_ref[...]) + b_ref[...]
        m = jnp.max(z, axis=1, keepdims=True)
        zz = z - m
        out_ref[...] = zz - jnp.log(jnp.sum(jnp.exp(zz), axis=1, keepdims=True))

    return pl.pallas_call(
        body,
        grid=(n_pad // RB,),
        in_specs=[
            pl.BlockSpec((RB, d), lambda i: (i, 0)),
            pl.BlockSpec((RB, d), lambda i: (i, 0)),
            pl.BlockSpec((RB, d), lambda i: (i, 0)),
            pl.BlockSpec((RB, 1), lambda i: (i, 0)),
            pl.BlockSpec((d,), lambda i: (0,)),
        ],
        out_specs=pl.BlockSpec((RB, d), lambda i: (i, 0)),
        out_shape=jax.ShapeDtypeStruct((n_pad, d), jnp.float32),
    )(q0, q1, hh, dinv2, b)


def kernel(x, edge_index, W1, b1, W2, b2):
    N, d_in = x.shape
    E = edge_index.shape[1]
    n_pad = (N // RB + 1) * RB

    src = edge_index[0].astype(jnp.int32)
    dst = edge_index[1].astype(jnp.int32)

    n_ch_h = -(-E // (NW * CHH))
    pad_h = NW * CHH * n_ch_h - E
    dsts_h = jnp.concatenate([dst, jnp.full((pad_h,), N, jnp.int32)])
    dsts_h = dsts_h.reshape(NW, n_ch_h, CHH)

    n_ch_p = -(-(-(-E // (NW * CH))) // NIS) * NIS
    pad_p = NW * CH * n_ch_p - E
    srcs_p = jnp.concatenate([src, jnp.zeros((pad_p,), jnp.int32)])
    dsts_p = jnp.concatenate([dst, jnp.full((pad_p,), N, jnp.int32)])
    srcs_p = srcs_p.reshape(NW, n_ch_p, CH)
    dsts_p = dsts_p.reshape(NW, n_ch_p, CH)

    x_pad = jnp.pad(x, ((0, n_pad - N), (0, 0)))
    zrows = jnp.zeros((CH, 128), jnp.float32)
    zeros1 = jnp.zeros((n_pad,), jnp.float32)

    hist = _hist_kernel(n_pad, n_ch_h)(dsts_h, zeros1)
    dinv2 = _dinv(hist)[:, None]

    prop = _prop_kernel(n_pad, n_ch_p)
    h1h = _mm_scale(x_pad, W1, dinv2)
    P = prop(h1h, srcs_p, dsts_p, zrows)
    h2h = _mid_layer(P[0], P[1], h1h, dinv2, b1, W2)
    Q = prop(h2h, srcs_p, dsts_p, zrows)
    out = _final_layer(Q[0], Q[1], h2h, dinv2, b2)
    return out[:N]


# speedup vs baseline: 31.6837x; 1.7852x over previous
import functools

import jax
import jax.numpy as jnp
from jax import lax
from jax.experimental import pallas as pl
from jax.experimental.pallas import tpu as pltpu
from jax.experimental.pallas import tpu_sc as plsc

NC = 2
NS = 16
LANES = 16
NW = NC * NS
CH = 120
CHH = 128
NSL = 3
NIS = 6
RB = 1024


def _sc_mesh():
    return plsc.VectorSubcoreMesh(core_axis_name="c", subcore_axis_name="s")


@functools.lru_cache(maxsize=None)
def _hist_kernel(n_pad, n_chunks):
    @functools.partial(
        pl.kernel,
        out_type=jax.ShapeDtypeStruct((NW, n_pad), jnp.float32),
        mesh=_sc_mesh(),
        scratch_types=[
            pltpu.VMEM((n_chunks, CHH), jnp.int32),
            pltpu.VMEM((n_pad,), jnp.float32),
        ],
        compiler_params=pltpu.CompilerParams(needs_layout_passes=False),
    )
    def hist(dsts_hbm, zeros1_hbm, out_hbm, idx_v, hist_v):
        c = lax.axis_index("c")
        s = lax.axis_index("s")
        w = c * NS + s
        pltpu.sync_copy(dsts_hbm.at[w], idx_v)
        pltpu.sync_copy(zeros1_hbm, hist_v)
        ones = jnp.ones((LANES,), jnp.float32)

        def body(g, carry):
            for j in range(CHH // LANES):
                idx = idx_v[g, pl.ds(j * LANES, LANES)]
                plsc.addupdate_scatter(hist_v, [idx], ones)
            return carry

        lax.fori_loop(0, n_chunks, body, 0)
        pltpu.sync_copy(hist_v, out_hbm.at[w])

    return hist


@functools.lru_cache(maxsize=None)
def _prop_kernel(n_pad, n_chunks):
    rows_per_tile = n_pad // NS
    wb_sizes = [CH] * (rows_per_tile // CH)
    if rows_per_tile % CH:
        wb_sizes.append(rows_per_tile % CH)

    @functools.partial(
        pl.kernel,
        out_type=jax.ShapeDtypeStruct((NC, n_pad, 128), jnp.float32),
        mesh=_sc_mesh(),
        scratch_types=(
            [pltpu.VMEM((CH,), jnp.int32) for _ in range(2 * NIS)]
            + [pltpu.VMEM((CH, 128), jnp.float32) for _ in range(NSL)]
            + [pltpu.VMEM_SHARED((n_pad, 128), jnp.float32)]
            + [pltpu.SemaphoreType.DMA] * (NIS + 2 * NSL)
        ),
        compiler_params=pltpu.CompilerParams(needs_layout_passes=False),
    )
    def prop(h_hbm, srcs_hbm, dsts_hbm, zrows_hbm, out_hbm, *scr):
        srcb = scr[0:NIS]
        dstb = scr[NIS:2 * NIS]
        bufs = scr[2 * NIS:2 * NIS + NSL]
        acc_sh = scr[2 * NIS + NSL]
        semi = scr[2 * NIS + NSL + 1:2 * NIS + NSL + 1 + NIS]
        semg = scr[2 * NIS + NSL + 1 + NIS:2 * NIS + NSL + 1 + NIS + NSL]
        semw = scr[2 * NIS + NSL + 1 + NIS + NSL:]
        c = lax.axis_index("c")
        s = lax.axis_index("s")
        w = c * NS + s

        def start_idx(g, k):
            pltpu.make_async_copy(srcs_hbm.at[w, g], srcb[k], semi[k]).start()
            pltpu.make_async_copy(dsts_hbm.at[w, g], dstb[k], semi[k]).start()

        def wait_idx(g, k):
            pltpu.make_async_copy(srcs_hbm.at[w, g], srcb[k], semi[k]).wait()
            pltpu.make_async_copy(dsts_hbm.at[w, g], dstb[k], semi[k]).wait()

        def start_gather(p, k):
            pltpu.make_async_copy(h_hbm.at[srcb[k]], bufs[p], semg[p]).start()

        def wait_gather(p):
            pltpu.make_async_copy(h_hbm.at[srcb[0]], bufs[p], semg[p]).wait()

        def start_scatter(p, k):
            pltpu.async_copy(bufs[p], acc_sh.at[dstb[k]], semw[p], add=True)

        def wait_scatter(p):
            pltpu.make_async_copy(bufs[p], acc_sh.at[dstb[0]], semw[p]).wait()

        pltpu.sync_copy(zrows_hbm, bufs[0])
        base = s * rows_per_tile
        off = 0
        for sz in wb_sizes:
            pltpu.sync_copy(bufs[0].at[pl.ds(0, sz)],
                            acc_sh.at[pl.ds(base + off, sz)])
            off += sz
        plsc.subcore_barrier()

        for k in range(min(NIS, n_chunks)):
            start_idx(k, k)
        wait_idx(0, 0)
        start_gather(0, 0)
        wait_idx(1, 1)
        start_gather(1, 1)

        def body(i, carry):
            for j in range(NIS):
                g = NIS * i + j
                p = j % NSL
                q = (j + 2) % NSL
                ki = (j + 2) % NIS
                kr = (j + NIS - 1) % NIS
                wait_gather(p)
                start_scatter(p, j)

                @pl.when(g >= 1)
                def _():
                    wait_scatter(q)

                @pl.when((g >= 1) & (g + NIS - 1 < n_chunks))
                def _():
                    start_idx(g + NIS - 1, kr)

                @pl.when(g + 2 < n_chunks)
                def _():
                    wait_idx(g + 2, ki)
                    start_gather(q, ki)

            return carry

        lax.fori_loop(0, n_chunks // NIS, body, 0)
        wait_scatter((n_chunks - 1) % NSL)
        plsc.subcore_barrier()

        off = 0
        for sz in wb_sizes:
            r = base + off
            pltpu.sync_copy(acc_sh.at[pl.ds(r, sz)], bufs[0].at[pl.ds(0, sz)])
            pltpu.sync_copy(bufs[0].at[pl.ds(0, sz)],
                            out_hbm.at[c, pl.ds(r, sz)])
            off += sz

    return prop


def _dinv(hist):
    nw, n_pad = hist.shape

    def body(hist_ref, out_ref):
        deg = 1.0 + jnp.sum(hist_ref[...], axis=0)
        out_ref[...] = lax.rsqrt(deg)

    return pl.pallas_call(
        body,
        grid=(n_pad // RB,),
        in_specs=[pl.BlockSpec((nw, RB), lambda i: (0, i))],
        out_specs=pl.BlockSpec((RB,), lambda i: (i,)),
        out_shape=jax.ShapeDtypeStruct((n_pad,), jnp.float32),
    )(hist)


def _mm_scale(x, W, dinv2):
    n_pad, d_in = x.shape
    d_out = W.shape[1]

    def body(x_ref, w_ref, dv_ref, out_ref):
        h = jnp.dot(x_ref[...], w_ref[...], preferred_element_type=jnp.float32)
        out_ref[...] = h * dv_ref[...]

    return pl.pallas_call(
        body,
        grid=(n_pad // RB,),
        in_specs=[
            pl.BlockSpec((RB, d_in), lambda i: (i, 0)),
            pl.BlockSpec((d_in, d_out), lambda i: (0, 0)),
            pl.BlockSpec((RB, 1), lambda i: (i, 0)),
        ],
        out_specs=pl.BlockSpec((RB, d_out), lambda i: (i, 0)),
        out_shape=jax.ShapeDtypeStruct((n_pad, d_out), jnp.float32),
    )(x, W, dinv2)


def _mid_layer(p0, p1, hh, dinv2, b, W):
    n_pad, d = hh.shape
    d_out = W.shape[1]

    def body(p0_ref, p1_ref, hh_ref, dv_ref, b_ref, w_ref, out_ref):
        z = dv_ref[...] * (p0_ref[...] + p1_ref[...] + hh_ref[...]) + b_ref[...]
        h1 = jnp.maximum(z, 0.0)
        h = jnp.dot(h1, w_ref[...], preferred_element_type=jnp.float32)
        out_ref[...] = h * dv_ref[...]

    return pl.pallas_call(
        body,
        grid=(n_pad // RB,),
        in_specs=[
            pl.BlockSpec((RB, d), lambda i: (i, 0)),
            pl.BlockSpec((RB, d), lambda i: (i, 0)),
            pl.BlockSpec((RB, d), lambda i: (i, 0)),
            pl.BlockSpec((RB, 1), lambda i: (i, 0)),
            pl.BlockSpec((d,), lambda i: (0,)),
            pl.BlockSpec((d, d_out), lambda i: (0, 0)),
        ],
        out_specs=pl.BlockSpec((RB, d_out), lambda i: (i, 0)),
        out_shape=jax.ShapeDtypeStruct((n_pad, d_out), jnp.float32),
    )(p0, p1, hh, dinv2, b, W)


def _final_layer(q0, q1, hh, dinv2, b):
    n_pad, d = hh.shape

    def body(q0_ref, q1_ref, hh_ref, dv_ref, b_ref, out_ref):
        z = dv_ref[...] * (q0_ref[...] + q1_ref[...] + hh_ref[...]) + b_ref[...]
        m = jnp.max(z, axis=1, keepdims=True)
        zz = z - m
        out_ref[...] = zz - jnp.log(jnp.sum(jnp.exp(zz), axis=1, keepdims=True))

    return pl.pallas_call(
        body,
        grid=(n_pad // RB,),
        in_specs=[
            pl.BlockSpec((RB, d), lambda i: (i, 0)),
            pl.BlockSpec((RB, d), lambda i: (i, 0)),
            pl.BlockSpec((RB, d), lambda i: (i, 0)),
            pl.BlockSpec((RB, 1), lambda i: (i, 0)),
            pl.BlockSpec((d,), lambda i: (0,)),
        ],
        out_specs=pl.BlockSpec((RB, d), lambda i: (i, 0)),
        out_shape=jax.ShapeDtypeStruct((n_pad, d), jnp.float32),
    )(q0, q1, hh, dinv2, b)


def kernel(x, edge_index, W1, b1, W2, b2):
    N, d_in = x.shape
    E = edge_index.shape[1]
    n_pad = (N // RB + 1) * RB

    src = edge_index[0].astype(jnp.int32)
    dst = edge_index[1].astype(jnp.int32)

    n_ch_h = -(-E // (NW * CHH))
    pad_h = NW * CHH * n_ch_h - E
    dsts_h = jnp.concatenate([dst, jnp.full((pad_h,), N, jnp.int32)])
    dsts_h = dsts_h.reshape(NW, n_ch_h, CHH)

    n_ch_p = -(-(-(-E // (NW * CH))) // NIS) * NIS
    pad_p = NW * CH * n_ch_p - E
    pad_ids = jnp.arange(pad_p, dtype=jnp.int32)
    srcs_p = jnp.concatenate([src, pad_ids % N])
    dsts_p = jnp.concatenate([dst, N + pad_ids % (n_pad - N)])
    srcs_p = srcs_p.reshape(NW, n_ch_p, CH)
    dsts_p = dsts_p.reshape(NW, n_ch_p, CH)

    x_pad = jnp.pad(x, ((0, n_pad - N), (0, 0)))
    zrows = jnp.zeros((CH, 128), jnp.float32)
    zeros1 = jnp.zeros((n_pad,), jnp.float32)

    hist = _hist_kernel(n_pad, n_ch_h)(dsts_h, zeros1)
    dinv2 = _dinv(hist)[:, None]

    prop = _prop_kernel(n_pad, n_ch_p)
    h1h = _mm_scale(x_pad, W1, dinv2)
    P = prop(h1h, srcs_p, dsts_p, zrows)
    h2h = _mid_layer(P[0], P[1], h1h, dinv2, b1, W2)
    Q = prop(h2h, srcs_p, dsts_p, zrows)
    out = _final_layer(Q[0], Q[1], h2h, dinv2, b2)
    return out[:N]


# speedup vs baseline: 32.1251x; 1.0139x over previous
import functools

import jax
import jax.numpy as jnp
from jax import lax
from jax.experimental import pallas as pl
from jax.experimental.pallas import tpu as pltpu
from jax.experimental.pallas import tpu_sc as plsc

NC = 2
NS = 16
LANES = 16
NW = NC * NS
CH = 120
NSL = 3
NIS = 6
RB = 1024
RBN = 1000


def _sc_mesh():
    return plsc.VectorSubcoreMesh(core_axis_name="c", subcore_axis_name="s")


@functools.lru_cache(maxsize=None)
def _hist_kernel(n_pad, e_per_tile):
    n_vecs = e_per_tile // LANES

    @functools.partial(
        pl.kernel,
        out_type=jax.ShapeDtypeStruct((NW, n_pad), jnp.float32),
        mesh=_sc_mesh(),
        scratch_types=[
            pltpu.VMEM((e_per_tile,), jnp.int32),
            pltpu.VMEM((n_pad,), jnp.float32),
        ],
        compiler_params=pltpu.CompilerParams(needs_layout_passes=False),
    )
    def hist(dsts_hbm, zeros1_hbm, out_hbm, idx_v, hist_v):
        c = lax.axis_index("c")
        s = lax.axis_index("s")
        w = c * NS + s
        pltpu.sync_copy(dsts_hbm.at[w], idx_v)
        pltpu.sync_copy(zeros1_hbm, hist_v)
        ones = jnp.ones((LANES,), jnp.float32)

        def body(k, carry):
            idx = idx_v[pl.ds(k * LANES, LANES)]
            plsc.addupdate_scatter(hist_v, [idx], ones)
            return carry

        lax.fori_loop(0, n_vecs, body, 0)
        pltpu.sync_copy(hist_v, out_hbm.at[w])

    return hist


@functools.lru_cache(maxsize=None)
def _prop_kernel(n_pad, n_chunks):
    rows_per_tile = n_pad // NS
    wb_sizes = [CH] * (rows_per_tile // CH)
    if rows_per_tile % CH:
        wb_sizes.append(rows_per_tile % CH)

    @functools.partial(
        pl.kernel,
        out_type=jax.ShapeDtypeStruct((NC, n_pad, 128), jnp.float32),
        mesh=_sc_mesh(),
        scratch_types=(
            [pltpu.VMEM((CH,), jnp.int32) for _ in range(2 * NIS)]
            + [pltpu.VMEM((CH, 128), jnp.float32) for _ in range(NSL)]
            + [pltpu.VMEM_SHARED((n_pad, 128), jnp.float32)]
            + [pltpu.SemaphoreType.DMA] * (NIS + 2 * NSL)
        ),
        compiler_params=pltpu.CompilerParams(needs_layout_passes=False),
    )
    def prop(h_hbm, srcs_hbm, dsts_hbm, zrows_hbm, out_hbm, *scr):
        srcb = scr[0:NIS]
        dstb = scr[NIS:2 * NIS]
        bufs = scr[2 * NIS:2 * NIS + NSL]
        acc_sh = scr[2 * NIS + NSL]
        semi = scr[2 * NIS + NSL + 1:2 * NIS + NSL + 1 + NIS]
        semg = scr[2 * NIS + NSL + 1 + NIS:2 * NIS + NSL + 1 + NIS + NSL]
        semw = scr[2 * NIS + NSL + 1 + NIS + NSL:]
        c = lax.axis_index("c")
        s = lax.axis_index("s")
        w = c * NS + s

        def start_idx(g, k):
            pltpu.make_async_copy(srcs_hbm.at[w, g], srcb[k], semi[k]).start()
            pltpu.make_async_copy(dsts_hbm.at[w, g], dstb[k], semi[k]).start()

        def wait_idx(g, k):
            pltpu.make_async_copy(srcs_hbm.at[w, g], srcb[k], semi[k]).wait()
            pltpu.make_async_copy(dsts_hbm.at[w, g], dstb[k], semi[k]).wait()

        def start_gather(p, k):
            pltpu.make_async_copy(h_hbm.at[srcb[k]], bufs[p], semg[p]).start()

        def wait_gather(p):
            pltpu.make_async_copy(h_hbm.at[srcb[0]], bufs[p], semg[p]).wait()

        def start_scatter(p, k):
            pltpu.async_copy(bufs[p], acc_sh.at[dstb[k]], semw[p], add=True)

        def wait_scatter(p):
            pltpu.make_async_copy(bufs[p], acc_sh.at[dstb[0]], semw[p]).wait()

        pltpu.sync_copy(zrows_hbm, bufs[0])
        base = s * rows_per_tile
        off = 0
        for sz in wb_sizes:
            pltpu.sync_copy(bufs[0].at[pl.ds(0, sz)],
                            acc_sh.at[pl.ds(base + off, sz)])
            off += sz
        plsc.subcore_barrier()

        for k in range(min(NIS, n_chunks)):
            start_idx(k, k)
        wait_idx(0, 0)
        start_gather(0, 0)
        wait_idx(1, 1)
        start_gather(1, 1)

        def body(i, carry):
            for j in range(NIS):
                g = NIS * i + j
                p = j % NSL
                q = (j + 2) % NSL
                ki = (j + 2) % NIS
                kr = (j + NIS - 1) % NIS
                wait_gather(p)
                start_scatter(p, j)

                @pl.when(g >= 1)
                def _():
                    wait_scatter(q)

                @pl.when((g >= 1) & (g + NIS - 1 < n_chunks))
                def _():
                    start_idx(g + NIS - 1, kr)

                @pl.when(g + 2 < n_chunks)
                def _():
                    wait_idx(g + 2, ki)
                    start_gather(q, ki)

            return carry

        lax.fori_loop(0, n_chunks // NIS, body, 0)
        wait_scatter((n_chunks - 1) % NSL)
        plsc.subcore_barrier()

        off = 0
        for sz in wb_sizes:
            r = base + off
            pltpu.sync_copy(acc_sh.at[pl.ds(r, sz)], bufs[0].at[pl.ds(0, sz)])
            pltpu.sync_copy(bufs[0].at[pl.ds(0, sz)],
                            out_hbm.at[c, pl.ds(r, sz)])
            off += sz

    return prop


def _dinv(hist):
    nw, n_pad = hist.shape

    def body(hist_ref, out_ref):
        deg = 1.0 + jnp.sum(hist_ref[...], axis=0)
        out_ref[...] = lax.rsqrt(deg)

    return pl.pallas_call(
        body,
        grid=(n_pad // RB,),
        in_specs=[pl.BlockSpec((nw, RB), lambda i: (0, i))],
        out_specs=pl.BlockSpec((RB,), lambda i: (i,)),
        out_shape=jax.ShapeDtypeStruct((n_pad,), jnp.float32),
    )(hist)


def _mm_scale(x, W, dinv2):
    n_pad, d_in = x.shape
    d_out = W.shape[1]

    def body(x_ref, w_ref, dv_ref, out_ref):
        h = jnp.dot(x_ref[...], w_ref[...], preferred_element_type=jnp.float32)
        out_ref[...] = h * dv_ref[...]

    return pl.pallas_call(
        body,
        grid=(n_pad // RBN,),
        in_specs=[
            pl.BlockSpec((RBN, d_in), lambda i: (i, 0)),
            pl.BlockSpec((d_in, d_out), lambda i: (0, 0)),
            pl.BlockSpec((RBN, 1), lambda i: (i, 0)),
        ],
        out_specs=pl.BlockSpec((RBN, d_out), lambda i: (i, 0)),
        out_shape=jax.ShapeDtypeStruct((n_pad, d_out), jnp.float32),
    )(x, W, dinv2)


def _mid_layer(p0, p1, hh, dinv2, b, W):
    n_rows, d = hh.shape
    d_out = W.shape[1]

    def body(p0_ref, p1_ref, hh_ref, dv_ref, b_ref, w_ref, out_ref):
        z = dv_ref[...] * (p0_ref[...] + p1_ref[...] + hh_ref[...]) + b_ref[...]
        h1 = jnp.maximum(z, 0.0)
        h = jnp.dot(h1, w_ref[...], preferred_element_type=jnp.float32)
        out_ref[...] = h * dv_ref[...]

    return pl.pallas_call(
        body,
        grid=(n_rows // RBN,),
        in_specs=[
            pl.BlockSpec((RBN, d), lambda i: (i, 0)),
            pl.BlockSpec((RBN, d), lambda i: (i, 0)),
            pl.BlockSpec((RBN, d), lambda i: (i, 0)),
            pl.BlockSpec((RBN, 1), lambda i: (i, 0)),
            pl.BlockSpec((d,), lambda i: (0,)),
            pl.BlockSpec((d, d_out), lambda i: (0, 0)),
        ],
        out_specs=pl.BlockSpec((RBN, d_out), lambda i: (i, 0)),
        out_shape=jax.ShapeDtypeStruct((n_rows, d_out), jnp.float32),
    )(p0, p1, hh, dinv2, b, W)


def _final_layer(q0, q1, hh, dinv2, b):
    n_rows, d = hh.shape

    def body(q0_ref, q1_ref, hh_ref, dv_ref, b_ref, out_ref):
        z = dv_ref[...] * (q0_ref[...] + q1_ref[...] + hh_ref[...]) + b_ref[...]
        m = jnp.max(z, axis=1, keepdims=True)
        zz = z - m
        out_ref[...] = zz - jnp.log(jnp.sum(jnp.exp(zz), axis=1, keepdims=True))

    return pl.pallas_call(
        body,
        grid=(n_rows // RBN,),
        in_specs=[
            pl.BlockSpec((RBN, d), lambda i: (i, 0)),
            pl.BlockSpec((RBN, d), lambda i: (i, 0)),
            pl.BlockSpec((RBN, d), lambda i: (i, 0)),
            pl.BlockSpec((RBN, 1), lambda i: (i, 0)),
            pl.BlockSpec((d,), lambda i: (0,)),
        ],
        out_specs=pl.BlockSpec((RBN, d), lambda i: (i, 0)),
        out_shape=jax.ShapeDtypeStruct((n_rows, d), jnp.float32),
    )(q0, q1, hh, dinv2, b)


def kernel(x, edge_index, W1, b1, W2, b2):
    N, d_in = x.shape
    E = edge_index.shape[1]
    n_pad = (N // RB + 1) * RB

    src = edge_index[0].astype(jnp.int32)
    dst = edge_index[1].astype(jnp.int32)

    n_ch_p = -(-(-(-E // (NW * CH))) // NIS) * NIS
    pad_p = NW * CH * n_ch_p - E
    pad_ids = jnp.arange(pad_p, dtype=jnp.int32)
    srcs_p = jnp.concatenate([src, pad_ids % N])
    dsts_p = jnp.concatenate([dst, N + pad_ids % (n_pad - N)])
    srcs_p = srcs_p.reshape(NW, n_ch_p, CH)
    dsts_p = dsts_p.reshape(NW, n_ch_p, CH)
    dsts_flat = dsts_p.reshape(NW, n_ch_p * CH)

    zrows = jnp.zeros((CH, 128), jnp.float32)
    zeros1 = jnp.zeros((n_pad,), jnp.float32)

    hist = _hist_kernel(n_pad, n_ch_p * CH)(dsts_flat, zeros1)
    dinv2 = _dinv(hist)[:N, None]

    prop = _prop_kernel(n_pad, n_ch_p)
    h1h = _mm_scale(x, W1, dinv2)
    P = prop(h1h, srcs_p, dsts_p, zrows)
    h2h = _mid_layer(P[0], P[1], h1h, dinv2, b1, W2)
    Q = prop(h2h, srcs_p, dsts_p, zrows)
    return _final_layer(Q[0], Q[1], h2h, dinv2, b2)


# speedup vs baseline: 32.9487x; 1.0256x over previous
import functools

import jax
import jax.numpy as jnp
from jax import lax
from jax.experimental import pallas as pl
from jax.experimental.pallas import tpu as pltpu
from jax.experimental.pallas import tpu_sc as plsc

NC = 2
NS = 16
LANES = 16
NW = NC * NS
CH = 120
NSL = 3
NIS = 6
RB = 1024
RBN = 1000


def _sc_mesh():
    return plsc.VectorSubcoreMesh(core_axis_name="c", subcore_axis_name="s")


@functools.lru_cache(maxsize=None)
def _hist_kernel(n_pad, e_per_tile):
    n_vecs = e_per_tile // LANES

    @functools.partial(
        pl.kernel,
        out_type=jax.ShapeDtypeStruct((NW, n_pad), jnp.float32),
        mesh=_sc_mesh(),
        scratch_types=[
            pltpu.VMEM((e_per_tile,), jnp.int32),
            pltpu.VMEM((n_pad,), jnp.float32),
        ],
        compiler_params=pltpu.CompilerParams(needs_layout_passes=False),
    )
    def hist(dsts_hbm, zeros1_hbm, out_hbm, idx_v, hist_v):
        c = lax.axis_index("c")
        s = lax.axis_index("s")
        w = c * NS + s
        pltpu.sync_copy(dsts_hbm.at[w], idx_v)
        pltpu.sync_copy(zeros1_hbm, hist_v)
        ones = jnp.ones((LANES,), jnp.float32)

        def body(k, carry):
            idx = idx_v[pl.ds(k * LANES, LANES)]
            plsc.addupdate_scatter(hist_v, [idx], ones)
            return carry

        lax.fori_loop(0, n_vecs, body, 0)
        pltpu.sync_copy(hist_v, out_hbm.at[w])

    return hist


@functools.lru_cache(maxsize=None)
def _prop_kernel(n_pad, n_chunks):
    rows_per_tile = n_pad // NS
    wb_sizes = [CH] * (rows_per_tile // CH)
    if rows_per_tile % CH:
        wb_sizes.append(rows_per_tile % CH)

    @functools.partial(
        pl.kernel,
        out_type=jax.ShapeDtypeStruct((NC, n_pad, 128), jnp.float32),
        mesh=_sc_mesh(),
        scratch_types=(
            [pltpu.VMEM((CH,), jnp.int32) for _ in range(2 * NIS)]
            + [pltpu.VMEM((CH, 128), jnp.float32) for _ in range(NSL)]
            + [pltpu.VMEM_SHARED((n_pad, 128), jnp.float32)]
            + [pltpu.SemaphoreType.DMA] * (NIS + 2 * NSL)
        ),
        compiler_params=pltpu.CompilerParams(needs_layout_passes=False),
    )
    def prop(h_hbm, srcs_hbm, dsts_hbm, zrows_hbm, out_hbm, *scr):
        srcb = scr[0:NIS]
        dstb = scr[NIS:2 * NIS]
        bufs = scr[2 * NIS:2 * NIS + NSL]
        acc_sh = scr[2 * NIS + NSL]
        semi = scr[2 * NIS + NSL + 1:2 * NIS + NSL + 1 + NIS]
        semg = scr[2 * NIS + NSL + 1 + NIS:2 * NIS + NSL + 1 + NIS + NSL]
        semw = scr[2 * NIS + NSL + 1 + NIS + NSL:]
        c = lax.axis_index("c")
        s = lax.axis_index("s")
        w = c * NS + s

        def start_idx(g, k):
            pltpu.make_async_copy(srcs_hbm.at[w, g], srcb[k], semi[k]).start()
            pltpu.make_async_copy(dsts_hbm.at[w, g], dstb[k], semi[k]).start()

        def wait_idx(g, k):
            pltpu.make_async_copy(srcs_hbm.at[w, g], srcb[k], semi[k]).wait()
            pltpu.make_async_copy(dsts_hbm.at[w, g], dstb[k], semi[k]).wait()

        def start_gather(p, k):
            pltpu.make_async_copy(h_hbm.at[srcb[k]], bufs[p], semg[p]).start()

        def wait_gather(p):
            pltpu.make_async_copy(h_hbm.at[srcb[0]], bufs[p], semg[p]).wait()

        def start_scatter(p, k):
            pltpu.async_copy(bufs[p], acc_sh.at[dstb[k]], semw[p], add=True)

        def wait_scatter(p):
            pltpu.make_async_copy(bufs[p], acc_sh.at[dstb[0]], semw[p]).wait()

        pltpu.sync_copy(zrows_hbm, bufs[0])
        base = s * rows_per_tile
        off = 0
        for sz in wb_sizes:
            pltpu.sync_copy(bufs[0].at[pl.ds(0, sz)],
                            acc_sh.at[pl.ds(base + off, sz)])
            off += sz
        plsc.subcore_barrier()

        for k in range(min(NIS, n_chunks)):
            start_idx(k, k)
        wait_idx(0, 0)
        start_gather(0, 0)
        wait_idx(1, 1)
        start_gather(1, 1)

        def body(i, carry):
            for j in range(NIS):
                g = NIS * i + j
                p = j % NSL
                q = (j + 2) % NSL
                ki = (j + 2) % NIS
                kr = (j + NIS - 1) % NIS
                wait_gather(p)
                start_scatter(p, j)

                @pl.when(g >= 1)
                def _():
                    wait_scatter(q)

                @pl.when((g >= 1) & (g + NIS - 1 < n_chunks))
                def _():
                    start_idx(g + NIS - 1, kr)

                @pl.when(g + 2 < n_chunks)
                def _():
                    wait_idx(g + 2, ki)
                    start_gather(q, ki)

            return carry

        lax.fori_loop(0, n_chunks // NIS, body, 0)
        wait_scatter((n_chunks - 1) % NSL)
        plsc.subcore_barrier()

        off = 0
        for sz in wb_sizes:
            r = base + off
            pltpu.sync_copy(acc_sh.at[pl.ds(r, sz)], bufs[0].at[pl.ds(0, sz)])
            pltpu.sync_copy(bufs[0].at[pl.ds(0, sz)],
                            out_hbm.at[c, pl.ds(r, sz)])
            off += sz

    return prop


def _mm(x, W):
    n_rows, d_in = x.shape
    d_out = W.shape[1]

    def body(x_ref, w_ref, out_ref):
        out_ref[...] = jnp.dot(x_ref[...], w_ref[...],
                               preferred_element_type=jnp.float32)

    return pl.pallas_call(
        body,
        grid=(n_rows // RBN,),
        in_specs=[
            pl.BlockSpec((RBN, d_in), lambda i: (i, 0)),
            pl.BlockSpec((d_in, d_out), lambda i: (0, 0)),
        ],
        out_specs=pl.BlockSpec((RBN, d_out), lambda i: (i, 0)),
        out_shape=jax.ShapeDtypeStruct((n_rows, d_out), jnp.float32),
    )(x, W)


def _dinv_scale(g1, hist):
    n_rows, d = g1.shape
    nw, n_pad = hist.shape

    def body(h_ref, g_ref, out_ref, dv_ref):
        deg = 1.0 + jnp.sum(h_ref[...], axis=0)
        dinv = lax.rsqrt(deg)[:, None]
        dv_ref[...] = dinv
        out_ref[...] = g_ref[...] * dinv

    return pl.pallas_call(
        body,
        grid=(n_pad // RB,),
        in_specs=[
            pl.BlockSpec((nw, RB), lambda i: (0, i)),
            pl.BlockSpec((RB, d), lambda i: (i, 0)),
        ],
        out_specs=[
            pl.BlockSpec((RB, d), lambda i: (i, 0)),
            pl.BlockSpec((RB, 1), lambda i: (i, 0)),
        ],
        out_shape=[
            jax.ShapeDtypeStruct((n_rows, d), jnp.float32),
            jax.ShapeDtypeStruct((n_rows, 1), jnp.float32),
        ],
    )(hist, g1)


def _mid_layer(p0, p1, hh, dinv2, b, W):
    n_rows, d = hh.shape
    d_out = W.shape[1]

    def body(p0_ref, p1_ref, hh_ref, dv_ref, b_ref, w_ref, out_ref):
        z = dv_ref[...] * (p0_ref[...] + p1_ref[...] + hh_ref[...]) + b_ref[...]
        h1 = jnp.maximum(z, 0.0)
        h = jnp.dot(h1, w_ref[...], preferred_element_type=jnp.float32)
        out_ref[...] = h * dv_ref[...]

    return pl.pallas_call(
        body,
        grid=(n_rows // RBN,),
        in_specs=[
            pl.BlockSpec((RBN, d), lambda i: (i, 0)),
            pl.BlockSpec((RBN, d), lambda i: (i, 0)),
            pl.BlockSpec((RBN, d), lambda i: (i, 0)),
            pl.BlockSpec((RBN, 1), lambda i: (i, 0)),
            pl.BlockSpec((d,), lambda i: (0,)),
            pl.BlockSpec((d, d_out), lambda i: (0, 0)),
        ],
        out_specs=pl.BlockSpec((RBN, d_out), lambda i: (i, 0)),
        out_shape=jax.ShapeDtypeStruct((n_rows, d_out), jnp.float32),
    )(p0, p1, hh, dinv2, b, W)


def _final_layer(q0, q1, hh, dinv2, b):
    n_rows, d = hh.shape

    def body(q0_ref, q1_ref, hh_ref, dv_ref, b_ref, out_ref):
        z = dv_ref[...] * (q0_ref[...] + q1_ref[...] + hh_ref[...]) + b_ref[...]
        m = jnp.max(z, axis=1, keepdims=True)
        zz = z - m
        out_ref[...] = zz - jnp.log(jnp.sum(jnp.exp(zz), axis=1, keepdims=True))

    return pl.pallas_call(
        body,
        grid=(n_rows // RBN,),
        in_specs=[
            pl.BlockSpec((RBN, d), lambda i: (i, 0)),
            pl.BlockSpec((RBN, d), lambda i: (i, 0)),
            pl.BlockSpec((RBN, d), lambda i: (i, 0)),
            pl.BlockSpec((RBN, 1), lambda i: (i, 0)),
            pl.BlockSpec((d,), lambda i: (0,)),
        ],
        out_specs=pl.BlockSpec((RBN, d), lambda i: (i, 0)),
        out_shape=jax.ShapeDtypeStruct((n_rows, d), jnp.float32),
    )(q0, q1, hh, dinv2, b)


def kernel(x, edge_index, W1, b1, W2, b2):
    N, d_in = x.shape
    E = edge_index.shape[1]
    n_pad = (N // RB + 1) * RB

    src = edge_index[0].astype(jnp.int32)
    dst = edge_index[1].astype(jnp.int32)

    n_ch_p = -(-(-(-E // (NW * CH))) // NIS) * NIS
    pad_p = NW * CH * n_ch_p - E
    pad_ids = jnp.arange(pad_p, dtype=jnp.int32)
    srcs_p = jnp.concatenate([src, pad_ids % N])
    dsts_p = jnp.concatenate([dst, N + pad_ids % (n_pad - N)])
    srcs_p = srcs_p.reshape(NW, n_ch_p, CH)
    dsts_p = dsts_p.reshape(NW, n_ch_p, CH)
    dsts_flat = dsts_p.reshape(NW, n_ch_p * CH)

    zrows = jnp.zeros((CH, 128), jnp.float32)
    zeros1 = jnp.zeros((n_pad,), jnp.float32)

    hist = _hist_kernel(n_pad, n_ch_p * CH)(dsts_flat, zeros1)
    g1 = _mm(x, W1)
    h1h, dinv2 = _dinv_scale(g1, hist)

    prop = _prop_kernel(n_pad, n_ch_p)
    P = prop(h1h, srcs_p, dsts_p, zrows)
    h2h = _mid_layer(P[0], P[1], h1h, dinv2, b1, W2)
    Q = prop(h2h, srcs_p, dsts_p, zrows)
    return _final_layer(Q[0], Q[1], h2h, dinv2, b2)


# speedup vs baseline: 34.9520x; 1.0608x over previous
import functools

import jax
import jax.numpy as jnp
from jax import lax
from jax.experimental import pallas as pl
from jax.experimental.pallas import tpu as pltpu
from jax.experimental.pallas import tpu_sc as plsc

NC = 2
NS = 16
LANES = 16
NW = NC * NS
CH = 80
NSL = 4
NIS = 8
RB = 1024
RBN = 1000


def _sc_mesh():
    return plsc.VectorSubcoreMesh(core_axis_name="c", subcore_axis_name="s")


@functools.lru_cache(maxsize=None)
def _hist_kernel(n, e_per_tile):
    n_vecs = e_per_tile // LANES

    @functools.partial(
        pl.kernel,
        out_type=jax.ShapeDtypeStruct((NW, n), jnp.float32),
        mesh=_sc_mesh(),
        scratch_types=[
            pltpu.VMEM((e_per_tile,), jnp.int32),
            pltpu.VMEM((n,), jnp.float32),
        ],
        compiler_params=pltpu.CompilerParams(needs_layout_passes=False),
    )
    def hist(dst_hbm, zeros1_hbm, out_hbm, idx_v, hist_v):
        c = lax.axis_index("c")
        s = lax.axis_index("s")
        w = c * NS + s
        pltpu.sync_copy(
            dst_hbm.at[pl.ds(pl.multiple_of(w * e_per_tile, 8), e_per_tile)],
            idx_v)
        pltpu.sync_copy(zeros1_hbm, hist_v)
        ones = jnp.ones((LANES,), jnp.float32)

        def body(k, carry):
            idx = idx_v[pl.ds(k * LANES, LANES)]
            plsc.addupdate_scatter(hist_v, [idx], ones)
            return carry

        lax.fori_loop(0, n_vecs, body, 0)
        pltpu.sync_copy(hist_v, out_hbm.at[w])

    return hist


@functools.lru_cache(maxsize=None)
def _prop_kernel(n_acc, e_per_tile):
    rows_per_tile = n_acc // NS
    n_chunks = e_per_tile // CH
    wb_sizes = [CH] * (rows_per_tile // CH)
    if rows_per_tile % CH:
        wb_sizes.append(rows_per_tile % CH)
    n_loop = max(0, (n_chunks // NIS - 1) * NIS)
    while n_loop and n_loop + 2 > n_chunks:
        n_loop -= NIS

    @functools.partial(
        pl.kernel,
        out_type=[jax.ShapeDtypeStruct((n_acc, 128), jnp.float32),
                  jax.ShapeDtypeStruct((n_acc, 128), jnp.float32)],
        mesh=_sc_mesh(),
        scratch_types=(
            [pltpu.VMEM((CH,), jnp.int32) for _ in range(2 * NIS)]
            + [pltpu.VMEM((CH, 128), jnp.float32) for _ in range(NSL)]
            + [pltpu.VMEM_SHARED((n_acc, 128), jnp.float32)]
            + [pltpu.SemaphoreType.DMA] * (NIS + 2 * NSL)
        ),
        compiler_params=pltpu.CompilerParams(needs_layout_passes=False),
    )
    def prop(h_hbm, src_hbm, dst_hbm, zrows_hbm, out0_hbm, out1_hbm, *scr):
        srcb = scr[0:NIS]
        dstb = scr[NIS:2 * NIS]
        bufs = scr[2 * NIS:2 * NIS + NSL]
        acc_sh = scr[2 * NIS + NSL]
        semi = scr[2 * NIS + NSL + 1:2 * NIS + NSL + 1 + NIS]
        semg = scr[2 * NIS + NSL + 1 + NIS:2 * NIS + NSL + 1 + NIS + NSL]
        semw = scr[2 * NIS + NSL + 1 + NIS + NSL:]
        c = lax.axis_index("c")
        s = lax.axis_index("s")
        e_base = (c * NS + s) * e_per_tile

        def e_off(g):
            return pl.multiple_of(e_base + g * CH, 8)

        def start_idx(g, k):
            pltpu.make_async_copy(src_hbm.at[pl.ds(e_off(g), CH)],
                                  srcb[k], semi[k]).start()
            pltpu.make_async_copy(dst_hbm.at[pl.ds(e_off(g), CH)],
                                  dstb[k], semi[k]).start()

        def wait_idx(g, k):
            pltpu.make_async_copy(src_hbm.at[pl.ds(e_off(g), CH)],
                                  srcb[k], semi[k]).wait()
            pltpu.make_async_copy(dst_hbm.at[pl.ds(e_off(g), CH)],
                                  dstb[k], semi[k]).wait()

        def start_gather(p, k):
            pltpu.make_async_copy(h_hbm.at[srcb[k]], bufs[p], semg[p]).start()

        def wait_gather(p):
            pltpu.make_async_copy(h_hbm.at[srcb[0]], bufs[p], semg[p]).wait()

        def start_scatter(p, k):
            pltpu.async_copy(bufs[p], acc_sh.at[dstb[k]], semw[p], add=True)

        def wait_scatter(p):
            pltpu.make_async_copy(bufs[p], acc_sh.at[dstb[0]], semw[p]).wait()

        pltpu.sync_copy(zrows_hbm, bufs[0])
        base = s * rows_per_tile
        off = 0
        for sz in wb_sizes:
            pltpu.sync_copy(bufs[0].at[pl.ds(0, sz)],
                            acc_sh.at[pl.ds(base + off, sz)])
            off += sz
        plsc.subcore_barrier()

        for k in range(min(NIS, n_chunks)):
            start_idx(k, k)
        for k in range(min(NSL - 1, n_chunks)):
            wait_idx(k, k)
            start_gather(k, k)

        def chunk_step(g, j, static):
            p = j % NSL
            q = (j + NSL - 1) % NSL
            ki = (j + NSL - 1) % NIS
            kr = (j + NIS - 1) % NIS
            wait_gather(p)
            start_scatter(p, j)
            if static:
                if g >= 1:
                    wait_scatter(q)
                if 1 <= g and g + NIS - 1 < n_chunks:
                    start_idx(g + NIS - 1, kr)
                if g + NSL - 1 < n_chunks:
                    wait_idx(g + NSL - 1, ki)
                    start_gather(q, ki)
            else:
                @pl.when(g >= 1)
                def _():
                    wait_scatter(q)
                    start_idx(g + NIS - 1, kr)

                wait_idx(g + NSL - 1, ki)
                start_gather(q, ki)

        def body(i, carry):
            for j in range(NIS):
                chunk_step(NIS * i + j, j, False)
            return carry

        lax.fori_loop(0, n_loop // NIS, body, 0)
        for g in range(n_loop, n_chunks):
            chunk_step(g, g % NIS, True)
        wait_scatter((n_chunks - 1) % NSL)
        plsc.subcore_barrier()

        off = 0
        for sz in wb_sizes:
            r = base + off
            pltpu.sync_copy(acc_sh.at[pl.ds(r, sz)], bufs[0].at[pl.ds(0, sz)])

            @pl.when(c == 0)
            def _():
                pltpu.sync_copy(bufs[0].at[pl.ds(0, sz)],
                                out0_hbm.at[pl.ds(r, sz)])

            @pl.when(c == 1)
            def _():
                pltpu.sync_copy(bufs[0].at[pl.ds(0, sz)],
                                out1_hbm.at[pl.ds(r, sz)])

            off += sz

    return prop


def _mm(x, W):
    n_rows, d_in = x.shape
    d_out = W.shape[1]

    def body(x_ref, w_ref, out_ref):
        out_ref[...] = jnp.dot(x_ref[...], w_ref[...],
                               preferred_element_type=jnp.float32)

    return pl.pallas_call(
        body,
        grid=(n_rows // RBN,),
        in_specs=[
            pl.BlockSpec((RBN, d_in), lambda i: (i, 0)),
            pl.BlockSpec((d_in, d_out), lambda i: (0, 0)),
        ],
        out_specs=pl.BlockSpec((RBN, d_out), lambda i: (i, 0)),
        out_shape=jax.ShapeDtypeStruct((n_rows, d_out), jnp.float32),
    )(x, W)


def _dinv_scale(g1, hist):
    n_rows, d = g1.shape
    nw = hist.shape[0]
    grid = -(-n_rows // RB)

    def body(h_ref, g_ref, out_ref, dv_ref):
        deg = 1.0 + jnp.sum(h_ref[...], axis=0)
        dinv = lax.rsqrt(deg)[:, None]
        dv_ref[...] = dinv
        out_ref[...] = g_ref[...] * dinv

    return pl.pallas_call(
        body,
        grid=(grid,),
        in_specs=[
            pl.BlockSpec((nw, RB), lambda i: (0, i)),
            pl.BlockSpec((RB, d), lambda i: (i, 0)),
        ],
        out_specs=[
            pl.BlockSpec((RB, d), lambda i: (i, 0)),
            pl.BlockSpec((RB, 1), lambda i: (i, 0)),
        ],
        out_shape=[
            jax.ShapeDtypeStruct((n_rows, d), jnp.float32),
            jax.ShapeDtypeStruct((n_rows, 1), jnp.float32),
        ],
    )(hist, g1)


def _mid_layer(p0, p1, hh, dinv2, b, W):
    n_rows, d = hh.shape
    d_out = W.shape[1]

    def body(p0_ref, p1_ref, hh_ref, dv_ref, b_ref, w_ref, out_ref):
        z = dv_ref[...] * (p0_ref[...] + p1_ref[...] + hh_ref[...]) + b_ref[...]
        h1 = jnp.maximum(z, 0.0)
        h = jnp.dot(h1, w_ref[...], preferred_element_type=jnp.float32)
        out_ref[...] = h * dv_ref[...]

    return pl.pallas_call(
        body,
        grid=(n_rows // RBN,),
        in_specs=[
            pl.BlockSpec((RBN, d), lambda i: (i, 0)),
            pl.BlockSpec((RBN, d), lambda i: (i, 0)),
            pl.BlockSpec((RBN, d), lambda i: (i, 0)),
            pl.BlockSpec((RBN, 1), lambda i: (i, 0)),
            pl.BlockSpec((d,), lambda i: (0,)),
            pl.BlockSpec((d, d_out), lambda i: (0, 0)),
        ],
        out_specs=pl.BlockSpec((RBN, d_out), lambda i: (i, 0)),
        out_shape=jax.ShapeDtypeStruct((n_rows, d_out), jnp.float32),
    )(p0, p1, hh, dinv2, b, W)


def _final_layer(q0, q1, hh, dinv2, b):
    n_rows, d = hh.shape

    def body(q0_ref, q1_ref, hh_ref, dv_ref, b_ref, out_ref):
        z = dv_ref[...] * (q0_ref[...] + q1_ref[...] + hh_ref[...]) + b_ref[...]
        m = jnp.max(z, axis=1, keepdims=True)
        zz = z - m
        out_ref[...] = zz - jnp.log(jnp.sum(jnp.exp(zz), axis=1, keepdims=True))

    return pl.pallas_call(
        body,
        grid=(n_rows // RBN,),
        in_specs=[
            pl.BlockSpec((RBN, d), lambda i: (i, 0)),
            pl.BlockSpec((RBN, d), lambda i: (i, 0)),
            pl.BlockSpec((RBN, d), lambda i: (i, 0)),
            pl.BlockSpec((RBN, 1), lambda i: (i, 0)),
            pl.BlockSpec((d,), lambda i: (0,)),
        ],
        out_specs=pl.BlockSpec((RBN, d), lambda i: (i, 0)),
        out_shape=jax.ShapeDtypeStruct((n_rows, d), jnp.float32),
    )(q0, q1, hh, dinv2, b)


def kernel(x, edge_index, W1, b1, W2, b2):
    N, d_in = x.shape
    E = edge_index.shape[1]
    e_per_tile = E // NW

    src = edge_index[0].astype(jnp.int32)
    dst = edge_index[1].astype(jnp.int32)

    n_acc = (N // RB + 1) * RB
    zrows = jnp.zeros((CH, 128), jnp.float32)
    zeros1 = jnp.zeros((N,), jnp.float32)

    hist = _hist_kernel(N, e_per_tile)(dst, zeros1)
    g1 = _mm(x, W1)
    h1h, dinv2 = _dinv_scale(g1, hist)

    prop = _prop_kernel(n_acc, e_per_tile)
    p0, p1 = prop(h1h, src, dst, zrows)
    h2h = _mid_layer(p0, p1, h1h, dinv2, b1, W2)
    q0, q1 = prop(h2h, src, dst, zrows)
    return _final_layer(q0, q1, h2h, dinv2, b2)
